# baseline (speedup 1.0000x reference)
import jax
import jax.numpy as jnp
from jax.experimental import pallas as pl
from jax.experimental.pallas import tpu as pltpu

_SB = 64


def _body(pos_ref, prev_ref, cur_ref, out_ref):
    out_ref[...] = prev_ref[...]
    p = pos_ref[0]
    j = pl.program_id(0)

    @pl.when(p // _SB == j)
    def _():
        out_ref[:, :, pl.ds(jax.lax.rem(p, _SB), 1), :] = cur_ref[...]


def kernel(prev, cur, dim, idx):
    B1, B2, S, D = prev.shape
    pos = (idx - (dim - 1)).astype(jnp.int32)
    out = pl.pallas_call(
        _body,
        grid_spec=pltpu.PrefetchScalarGridSpec(
            num_scalar_prefetch=1,
            grid=(S // _SB,),
            in_specs=[
                pl.BlockSpec((B1, B2, _SB, D), lambda j, p: (0, 0, j, 0)),
                pl.BlockSpec((B1, B2, 1, D), lambda j, p: (0, 0, 0, 0)),
            ],
            out_specs=pl.BlockSpec((B1, B2, _SB, D), lambda j, p: (0, 0, j, 0)),
        ),
        out_shape=jax.ShapeDtypeStruct(prev.shape, prev.dtype),
        compiler_params=pltpu.CompilerParams(
            dimension_semantics=("arbitrary",),
        ),
    )(pos, prev, cur)
    return out


# speedup vs baseline: 1.4646x; 1.4646x over previous
import jax
import jax.numpy as jnp
from jax.experimental import pallas as pl
from jax.experimental.pallas import tpu as pltpu


def _body(pos_ref, prev_ref, cur_ref, out_ref, sem):
    del prev_ref
    p = pos_ref[0]
    cp = pltpu.make_async_copy(
        cur_ref, out_ref.at[:, :, pl.ds(p, 1), :], sem)
    cp.start()
    cp.wait()


def kernel(prev, cur, dim, idx):
    pos = (idx - (dim - 1)).astype(jnp.int32)
    out = pl.pallas_call(
        _body,
        grid_spec=pltpu.PrefetchScalarGridSpec(
            num_scalar_prefetch=1,
            grid=(1,),
            in_specs=[
                pl.BlockSpec(memory_space=pl.ANY),
                pl.BlockSpec(memory_space=pl.ANY),
            ],
            out_specs=pl.BlockSpec(memory_space=pl.ANY),
            scratch_shapes=[pltpu.SemaphoreType.DMA],
        ),
        out_shape=jax.ShapeDtypeStruct(prev.shape, prev.dtype),
        input_output_aliases={1: 0},
    )(pos, prev, cur)
    return out
